# baseline (speedup 1.0000x reference)
import functools

import jax
import jax.numpy as jnp
import numpy as np
from jax import lax
from jax.experimental import pallas as pl
from jax.experimental.pallas import tpu as pltpu
from jax.experimental.pallas import tpu_sc as plsc

N = 10000
D = 128
E = 320000

NUM_TILES = 16
NUM_WORKERS = 32
EDGES_PER_WORKER = E // NUM_WORKERS
CHUNK = 80
NFULL = EDGES_PER_WORKER // CHUNK
NPAIR = (NFULL - 1) // 2
ROWS_PER_TILE = 624
ZCHUNK = 48
NZ = ROWS_PER_TILE // ZCHUNK
TAIL_ROWS = N - NUM_TILES * ROWS_PER_TILE

_F32 = jnp.float32
_BF16 = jnp.bfloat16

_PERM = np.empty((D,), np.int32)
for _g in range(D // 32):
    for _i in range(16):
        _PERM[32 * _g + _i] = 32 * _g + 2 * _i
        _PERM[32 * _g + 16 + _i] = 32 * _g + 2 * _i + 1


def _hx_body(x_ref, ws_ref, bs_ref, wa_ref, ba_ref, hs_ref, ha_ref):
    xv = x_ref[...].astype(_BF16)
    hs_ref[...] = jnp.tanh(
        jnp.dot(xv, ws_ref[...], preferred_element_type=_F32) + bs_ref[...])
    ha_ref[...] = jnp.tanh(
        jnp.dot(xv, wa_ref[...], preferred_element_type=_F32) + ba_ref[...])


_BN = 2000

_hx_call = pl.pallas_call(
    _hx_body,
    grid=(N // _BN,),
    in_specs=[
        pl.BlockSpec((_BN, D), lambda i: (i, 0)),
        pl.BlockSpec((D, D), lambda i: (0, 0)),
        pl.BlockSpec((1, D), lambda i: (0, 0)),
        pl.BlockSpec((D, D), lambda i: (0, 0)),
        pl.BlockSpec((1, D), lambda i: (0, 0)),
    ],
    out_specs=[
        pl.BlockSpec((_BN, D), lambda i: (i, 0)),
        pl.BlockSpec((_BN, D), lambda i: (i, 0)),
    ],
    out_shape=[
        jax.ShapeDtypeStruct((N, D), _F32),
        jax.ShapeDtypeStruct((N, D), _F32),
    ],
)


def _we_body(f_ref, wu_ref, bu_ref, we_ref, be_ref, wo_ref, bo_ref, o_ref):
    fb = f_ref[...]
    f = jnp.tanh(
        jnp.dot(fb.astype(_BF16), wu_ref[...], preferred_element_type=_F32)
        + bu_ref[...]) + fb
    f16 = f.astype(_BF16)
    ve = jnp.tanh(jnp.dot(f16, we_ref[...], preferred_element_type=_F32)
                  + be_ref[...]).astype(_BF16).astype(_F32)
    vo = jnp.tanh(jnp.dot(f16, wo_ref[...], preferred_element_type=_F32)
                  + bo_ref[...]).astype(_BF16).astype(_F32)
    i32 = jnp.int32
    ebits = jax.lax.shift_right_logical(
        jax.lax.bitcast_convert_type(ve, i32), 16)
    obits = jax.lax.bitcast_convert_type(vo, i32) & jnp.int32(-65536)
    o_ref[...] = obits | ebits


_BE = 5000

_we_call = pl.pallas_call(
    _we_body,
    grid=(E // _BE,),
    in_specs=[
        pl.BlockSpec((_BE, D), lambda i: (i, 0)),
        pl.BlockSpec((D, D), lambda i: (0, 0)),
        pl.BlockSpec((1, D), lambda i: (0, 0)),
        pl.BlockSpec((D, D // 2), lambda i: (0, 0)),
        pl.BlockSpec((1, D // 2), lambda i: (0, 0)),
        pl.BlockSpec((D, D // 2), lambda i: (0, 0)),
        pl.BlockSpec((1, D // 2), lambda i: (0, 0)),
    ],
    out_specs=pl.BlockSpec((_BE, D // 2), lambda i: (i, 0)),
    out_shape=jax.ShapeDtypeStruct((E, D // 2), jnp.int32),
)


def _upd_body(x_ref, zs0_ref, zs1_ref, za0_ref, za1_ref, wg_ref, bg_ref,
              o_ref):
    xv = x_ref[...]
    acc = jnp.dot(xv, wg_ref[0:D, :], preferred_element_type=_F32)
    acc = acc + jnp.dot(zs0_ref[...] + zs1_ref[...], wg_ref[D:2 * D, :],
                        preferred_element_type=_F32)
    acc = acc + jnp.dot(za0_ref[...] + za1_ref[...], wg_ref[2 * D:3 * D, :],
                        preferred_element_type=_F32)
    o_ref[...] = xv + jnp.tanh(acc + bg_ref[...])


_upd_call = pl.pallas_call(
    _upd_body,
    grid=(N // _BN,),
    in_specs=[
        pl.BlockSpec((_BN, D), lambda i: (i, 0)),
        pl.BlockSpec((_BN, D), lambda i: (i, 0)),
        pl.BlockSpec((_BN, D), lambda i: (i, 0)),
        pl.BlockSpec((_BN, D), lambda i: (i, 0)),
        pl.BlockSpec((_BN, D), lambda i: (i, 0)),
        pl.BlockSpec((3 * D, D), lambda i: (0, 0)),
        pl.BlockSpec((1, D), lambda i: (0, 0)),
    ],
    out_specs=pl.BlockSpec((_BN, D), lambda i: (i, 0)),
    out_shape=jax.ShapeDtypeStruct((N, D), _F32),
)


@functools.partial(
    pl.kernel,
    out_type=(
        jax.ShapeDtypeStruct((N, D), _F32),
        jax.ShapeDtypeStruct((N, D), _F32),
    ),
    mesh=plsc.VectorSubcoreMesh(core_axis_name="c", subcore_axis_name="s"),
    scratch_types=[
        pltpu.VMEM((2, CHUNK), jnp.int32),
        pltpu.VMEM((2, CHUNK), jnp.int32),
        pltpu.VMEM((CHUNK, D), _F32),
        pltpu.VMEM((CHUNK, D), _F32),
        pltpu.VMEM((CHUNK, D // 2), jnp.int32),
        pltpu.VMEM((CHUNK, D // 2), jnp.int32),
        pltpu.VMEM((ZCHUNK, D), _F32),
        pltpu.VMEM_SHARED((N, D), _F32),
        pltpu.SemaphoreType.DMA,
        pltpu.SemaphoreType.DMA,
        pltpu.SemaphoreType.DMA,
        pltpu.SemaphoreType.DMA,
        pltpu.SemaphoreType.DMA,
        pltpu.SemaphoreType.DMA,
        pltpu.SemaphoreType.DMA,
        pltpu.SemaphoreType.DMA,
        pltpu.SemaphoreType.DMA,
    ],
)
def _sc_aggregate(hx_hbm, we_hbm, idx_hbm,
                  z0_hbm, z1_hbm, idx0, idx1, hx0, hx1,
                  we0, we1, zbuf, z_sh, gsem0, gsem1,
                  wsem0, wsem1, ssem0, ssem1, isem0, isem1, zwsem):
    c = lax.axis_index("c")
    s = lax.axis_index("s")
    w = c * NUM_TILES + s
    bufs = ((idx0, hx0, we0, gsem0, wsem0, ssem0, isem0),
            (idx1, hx1, we1, gsem1, wsem1, ssem1, isem1))

    zeros16 = jnp.zeros((16,), _F32)

    def _zrow(r, carry):
        for g in range(D // 16):
            zbuf[r, pl.ds(g * 16, 16)] = zeros16
        return carry

    lax.fori_loop(0, ZCHUNK, _zrow, 0)
    for j in range(NZ):
        pltpu.async_copy(
            zbuf, z_sh.at[pl.ds(s * ROWS_PER_TILE + j * ZCHUNK, ZCHUNK)],
            zwsem)

    @pl.when(s == 0)
    def _():
        pltpu.async_copy(zbuf.at[pl.ds(0, TAIL_ROWS)],
                         z_sh.at[pl.ds(NUM_TILES * ROWS_PER_TILE, TAIL_ROWS)],
                         zwsem)

    for j in range(NZ):
        pltpu.make_async_copy(
            zbuf, z_sh.at[pl.ds(s * ROWS_PER_TILE + j * ZCHUNK, ZCHUNK)],
            zwsem).wait()

    @pl.when(s == 0)
    def _():
        pltpu.make_async_copy(
            zbuf.at[pl.ds(0, TAIL_ROWS)],
            z_sh.at[pl.ds(NUM_TILES * ROWS_PER_TILE, TAIL_ROWS)],
            zwsem).wait()

    plsc.subcore_barrier()

    def _base(k):
        return w * EDGES_PER_WORKER + k * CHUNK

    def _start(k, b):
        ib, hxb, web, gsem, wsem, ssem, isem = bufs[b]
        base = _base(k)

        @pl.when(k >= 2)
        def _():
            pltpu.make_async_copy(hxb, z_sh.at[ib.at[1]], ssem).wait()

        pltpu.make_async_copy(
            idx_hbm.at[w * NFULL + k], ib, isem).wait()
        pltpu.async_copy(hx_hbm.at[ib.at[0]], hxb, gsem)
        pltpu.async_copy(we_hbm.at[pl.ds(base, CHUNK)], web, wsem)

    def _finish(k, b):
        ib, hxb, web, gsem, wsem, ssem, isem = bufs[b]
        base = _base(k)
        pltpu.make_async_copy(hx_hbm.at[ib.at[0]], hxb, gsem).wait()
        pltpu.make_async_copy(
            we_hbm.at[pl.ds(base, CHUNK)], web, wsem).wait()

        def _unpk(iv):
            lo = jax.lax.bitcast_convert_type(iv << 16, _F32)
            hi = jax.lax.bitcast_convert_type(iv & jnp.int32(-65536), _F32)
            return lo, hi

        @plsc.parallel_loop(0, CHUNK, unroll=8)
        def _mul(r):
            for g in range(D // 32):
                wa, wb = _unpk(web[r, pl.ds(g * 16, 16)])
                sl_a = pl.ds(g * 32, 16)
                sl_b = pl.ds(g * 32 + 16, 16)
                hxb[r, sl_a] = wa * hxb[r, sl_a]
                hxb[r, sl_b] = wb * hxb[r, sl_b]

        pltpu.async_copy(hxb, z_sh.at[ib.at[1]], ssem, add=True)

        @pl.when(k < NFULL - 2)
        def _():
            pltpu.async_copy(idx_hbm.at[w * NFULL + k + 2], ib, isem)

    pltpu.async_copy(idx_hbm.at[w * NFULL], idx0, isem0)
    pltpu.async_copy(idx_hbm.at[w * NFULL + 1], idx1, isem1)
    _start(0, 0)

    def _pair(i, carry):
        _start(2 * i + 1, 1)
        _finish(2 * i, 0)
        _start(2 * i + 2, 0)
        _finish(2 * i + 1, 1)
        return carry

    lax.fori_loop(0, NPAIR, _pair, 0)
    _finish(NFULL - 1, 0)
    pltpu.make_async_copy(hx1, z_sh.at[idx1.at[1]], ssem1).wait()
    pltpu.make_async_copy(hx0, z_sh.at[idx0.at[1]], ssem0).wait()
    plsc.subcore_barrier()

    def _writeout(z_out):
        sl = pl.ds(s * ROWS_PER_TILE, ROWS_PER_TILE)
        pltpu.sync_copy(z_sh.at[sl], z_out.at[sl])

        @pl.when(s == 0)
        def _():
            sl = pl.ds(NUM_TILES * ROWS_PER_TILE, TAIL_ROWS)
            pltpu.sync_copy(z_sh.at[sl], z_out.at[sl])

    @pl.when(c == 0)
    def _():
        _writeout(z0_hbm)

    @pl.when(c == 1)
    def _():
        _writeout(z1_hbm)


def kernel(x, feat_same, feat_anti, senders_same, receivers_same, senders_anti,
           receivers_anti, W_u_same, b_u_same, W_u_anti, b_u_anti, W_w_same,
           b_w_same, W_w_anti, b_w_anti, W_h_same, b_h_same, W_h_anti,
           b_h_anti, W_g, b_g):
    r = lambda b: b.reshape(1, D)
    i32 = jnp.int32
    perm = jnp.asarray(_PERM)
    W_g_fix = jnp.concatenate(
        [W_g[0:D], W_g[D:2 * D][perm], W_g[2 * D:3 * D][perm]], axis=0)
    bf = lambda a: a.astype(jnp.bfloat16)
    hx_s, hx_a = _hx_call(x, bf(W_h_same[:, perm]), r(b_h_same[perm]),
                          bf(W_h_anti[:, perm]), r(b_h_anti[perm]))
    r2 = lambda b: b.reshape(1, D // 2)
    we_s = _we_call(feat_same, bf(W_u_same), r(b_u_same),
                    bf(W_w_same[:, 0::2]), r2(b_w_same[0::2]),
                    bf(W_w_same[:, 1::2]), r2(b_w_same[1::2]))
    stk = lambda a, b: jnp.stack(
        [a.astype(i32).reshape(E // CHUNK, CHUNK),
         b.astype(i32).reshape(E // CHUNK, CHUNK)], axis=1)
    zs0, zs1 = _sc_aggregate(hx_s, we_s, stk(senders_same, receivers_same))
    we_a = _we_call(feat_anti, bf(W_u_anti), r(b_u_anti),
                    bf(W_w_anti[:, 0::2]), r2(b_w_anti[0::2]),
                    bf(W_w_anti[:, 1::2]), r2(b_w_anti[1::2]))
    za0, za1 = _sc_aggregate(hx_a, we_a, stk(senders_anti, receivers_anti))
    return _upd_call(x, zs0, zs1, za0, za1, W_g_fix, r(b_g))
